# Initial kernel scaffold; baseline (speedup 1.0000x reference)
import jax
import jax.numpy as jnp
from jax.experimental import pallas as pl


def kernel(x, edge_index, tconv_w, tconv_b, bn0_w, bn0_b, bn0_rm, bn0_rv, W1, att_src1, att_dst1, bias1, bn1_w, bn1_b, bn1_rm, bn1_rv, W2, att_src2, att_dst2, bias2):
    raise NotImplementedError("write your pallas kernel here")


# speedup vs baseline: 6.5273x; 6.5273x over previous
import functools
import jax
import jax.numpy as jnp
from jax import lax
from jax.experimental import pallas as pl
from jax.experimental.pallas import tpu as pltpu
from jax.experimental.pallas import tpu_sc as plsc

def _dyngather(vec16, idx16):
    dnums = lax.GatherDimensionNumbers(
        offset_dims=(), collapsed_slice_dims=(0,), start_index_map=(0,))
    return lax.gather(vec16, idx16[:, None], dnums, (1,),
                      mode=lax.GatherScatterMode.PROMISE_IN_BOUNDS)


N = 10000
NP = 10240
BN = 512
E = 160000
EPAD = 163840
IN_DIM = 256
WIN = 8
TC_CH = 16
HID = 256
HEADS = 8
OUT = 2
EPS = 1e-5
NBLK = 16
NTS = NP // 16


def _ka_body(x_ref, wc_ref, b0_ref, w1_ref, d1_ref, au_ref,
             hb_ref, u_ref, m1_ref, mx_scr):
    i = pl.program_id(0)
    z = jnp.dot(x_ref[...], wc_ref[...], preferred_element_type=jnp.float32)
    r = jnp.maximum(z + b0_ref[...], 0.0)
    h = jnp.dot(r, w1_ref[...], preferred_element_type=jnp.float32) + d1_ref[...]
    for j in range(NBLK):
        hb_ref[j] = h[:, j * 128:(j + 1) * 128]
    u = jnp.dot(h, au_ref[...], preferred_element_type=jnp.float32)
    u_ref[...] = u
    bmax = jnp.max(u, axis=0, keepdims=True)

    @pl.when(i == 0)
    def _():
        mx_scr[...] = bmax

    @pl.when(i > 0)
    def _():
        mx_scr[...] = jnp.maximum(mx_scr[...], bmax)

    @pl.when(i == pl.num_programs(0) - 1)
    def _():
        m = mx_scr[0:1, 0:16] + mx_scr[0:1, 16:32]
        m1_ref[...] = jnp.where(m > 0, m, 0.2 * m)


def _tc_a(xp, Wc, b0r, W1g, d1, AU):
    grid = NP // BN
    return pl.pallas_call(
        _ka_body,
        grid=(grid,),
        in_specs=[
            pl.BlockSpec((BN, IN_DIM), lambda i: (i, 0)),
            pl.BlockSpec((IN_DIM, 128), lambda i: (0, 0)),
            pl.BlockSpec((1, 128), lambda i: (0, 0)),
            pl.BlockSpec((128, 2048), lambda i: (0, 0)),
            pl.BlockSpec((1, 2048), lambda i: (0, 0)),
            pl.BlockSpec((2048, 128), lambda i: (0, 0)),
        ],
        out_specs=[
            pl.BlockSpec((NBLK, BN, 128), lambda i: (0, i, 0)),
            pl.BlockSpec((BN, 128), lambda i: (i, 0)),
            pl.BlockSpec((1, 16), lambda i: (0, 0)),
        ],
        out_shape=[
            jax.ShapeDtypeStruct((NBLK, NP, 128), jnp.float32),
            jax.ShapeDtypeStruct((NP, 128), jnp.float32),
            jax.ShapeDtypeStruct((1, 16), jnp.float32),
        ],
        scratch_shapes=[pltpu.VMEM((1, 128), jnp.float32)],
    )(xp, Wc, b0r, W1g, d1, AU)


def _kd_body(hb_ref, agg_ref, u_ref, s1p_ref, m1_ref, g1_ref,
             c1_ref, w2e_ref, m16_ref, h2_ref, m2_ref, ts_ref, td_ref,
             ws_scr, mx_scr):
    i = pl.program_id(0)
    j = pl.program_id(1)

    @pl.when(j == 0)
    def _():
        asrc = u_ref[:, 0:8]
        adst = u_ref[:, 16:24]
        l = asrc + adst
        l = jnp.where(l > 0, l, 0.2 * l)
        wself = jnp.exp(l - m1_ref[0:1, 0:8])
        s = s1p_ref[0, :, 0:8] + s1p_ref[1, :, 0:8] + wself
        ws_scr[:, 0:8] = wself
        ws_scr[:, 8:16] = 1.0 / s

    head = j // 2
    wsel = jnp.zeros((BN, 1), jnp.float32)
    rsel = jnp.zeros((BN, 1), jnp.float32)
    for hd in range(HEADS):
        pick = (head == hd)
        wsel = jnp.where(pick, ws_scr[:, hd:hd + 1], wsel)
        rsel = jnp.where(pick, ws_scr[:, 8 + hd:9 + hd], rsel)

    out1 = (agg_ref[0] + wsel * hb_ref[0]) * rsel
    t = out1 * g1_ref[0] + c1_ref[0]
    e = jnp.where(t > 0, t, jnp.exp(jnp.minimum(t, 0.0)) - 1.0)
    part = jnp.dot(e, w2e_ref[...], preferred_element_type=jnp.float32)

    @pl.when(j == 0)
    def _():
        h2_ref[...] = part

    @pl.when(j > 0)
    def _():
        h2_ref[...] = h2_ref[...] + part

    @pl.when(j == pl.num_programs(1) - 1)
    def _():
        t = h2_ref[...]
        il = lax.broadcasted_iota(jnp.int32, (1, 128), 1)
        ts_ref[...] = jnp.where(
            il == 0, t[:, 0:1], jnp.where(il == 1, t[:, 1:2], t[:, 2:3]))
        td_ref[...] = jnp.where(il < 2, 0.0, t[:, 3:4])
        bmax = jnp.max(h2_ref[...], axis=0, keepdims=True)

        @pl.when(i == 0)
        def _():
            mx_scr[...] = bmax

        @pl.when(i > 0)
        def _():
            mx_scr[...] = jnp.maximum(mx_scr[...], bmax)

        @pl.when(i == pl.num_programs(0) - 1)
        def _():
            m = jnp.dot(mx_scr[...], m16_ref[...],
                        preferred_element_type=jnp.float32)
            m2_ref[...] = jnp.where(m > 0, m, 0.2 * m)


def _tc_d(hb, aggb, U, s1p, m1t, g1b, c1b, W2ext, M16b):
    grid = (NP // BN, NBLK)
    return pl.pallas_call(
        _kd_body,
        grid=grid,
        in_specs=[
            pl.BlockSpec((1, BN, 128), lambda i, j: (j, i, 0)),
            pl.BlockSpec((1, BN, 128), lambda i, j: (j, i, 0)),
            pl.BlockSpec((BN, 128), lambda i, j: (i, 0)),
            pl.BlockSpec((2, BN, 16), lambda i, j: (0, i, 0)),
            pl.BlockSpec((1, 16), lambda i, j: (0, 0)),
            pl.BlockSpec((1, 1, 128), lambda i, j: (j, 0, 0)),
            pl.BlockSpec((1, 1, 128), lambda i, j: (j, 0, 0)),
            pl.BlockSpec((128, 16), lambda i, j: (j, 0)),
            pl.BlockSpec((16, 16), lambda i, j: (0, 0)),
        ],
        out_specs=[
            pl.BlockSpec((BN, 16), lambda i, j: (i, 0)),
            pl.BlockSpec((1, 16), lambda i, j: (0, 0)),
            pl.BlockSpec((BN, 128), lambda i, j: (i, 0)),
            pl.BlockSpec((BN, 128), lambda i, j: (i, 0)),
        ],
        out_shape=[
            jax.ShapeDtypeStruct((NP, 16), jnp.float32),
            jax.ShapeDtypeStruct((1, 16), jnp.float32),
            jax.ShapeDtypeStruct((NP, 128), jnp.float32),
            jax.ShapeDtypeStruct((NP, 128), jnp.float32),
        ],
        scratch_shapes=[
            pltpu.VMEM((BN, 16), jnp.float32),
            pltpu.VMEM((1, 16), jnp.float32),
        ],
    )(hb, aggb, U, s1p, m1t, g1b, c1b, W2ext, M16b)


_EPT = EPAD // 32
_CHB = 128


def _sc_b(src, dst, U, m1t, zb):
    mesh = plsc.VectorSubcoreMesh(core_axis_name="c", subcore_axis_name="s")

    @functools.partial(
        pl.kernel, mesh=mesh,
        out_type=[jax.ShapeDtypeStruct((EPAD, 16), jnp.float32),
                  jax.ShapeDtypeStruct((2, NP // 8, 128), jnp.float32)],
        scratch_types=[
            pltpu.VMEM((_CHB,), jnp.int32),
            pltpu.VMEM((_CHB,), jnp.int32),
            pltpu.VMEM((_CHB,), jnp.int32),
            pltpu.VMEM((_CHB // 16, 16), jnp.float32),
            pltpu.VMEM((_CHB, 128), jnp.float32),
            pltpu.VMEM((_CHB, 128), jnp.float32),
            pltpu.VMEM((_CHB, 16), jnp.float32),
            pltpu.VMEM((_CHB, 128), jnp.float32),
            pltpu.VMEM((16,), jnp.float32),
            pltpu.VMEM_SHARED((NP // 8, 128), jnp.float32),
            pltpu.SemaphoreType.DMA,
        ],
    )
    def kb(src_h, dst_h, u_h, m1_h, zb_h, w_h, s1_h,
           si_v, di_v, dip_v, off_v, ga_v, gd_v, w_v, stg_v, m1_v,
           s1_sh, sem):
        cid = lax.axis_index("c")
        sid = lax.axis_index("s")
        wid = sid * 2 + cid
        nps = NP // 128
        pltpu.sync_copy(zb_h, s1_sh.at[pl.ds(sid * nps, nps)])
        pltpu.sync_copy(m1_h.at[0], m1_v)
        plsc.subcore_barrier()
        m1 = m1_v[...]
        base = wid * _EPT

        def chunk(ci, _):
            off = base + ci * _CHB
            pltpu.sync_copy(src_h.at[pl.ds(off, _CHB)], si_v)
            pltpu.sync_copy(dst_h.at[pl.ds(off, _CHB)], di_v)
            pltpu.async_copy(u_h.at[si_v], ga_v, sem).wait()
            pltpu.async_copy(u_h.at[di_v], gd_v, sem).wait()

            def pre(g, _):
                dvec = di_v[pl.ds(g * 16, 16)]
                off_v[g, :] = (dvec & 7).astype(jnp.float32)
                dip_v[pl.ds(g * 16, 16)] = lax.shift_right_logical(dvec, 3)
                return 0

            lax.fori_loop(0, _CHB // 16, pre, 0)

            def row(k, _):
                l = ga_v[k, pl.ds(0, 16)] + gd_v[k, pl.ds(16, 16)]
                l = jnp.where(l > 0, l, 0.2 * l)
                w = jnp.exp(l - m1)
                w_v[k, :] = w
                offb = _dyngather(off_v[k >> 4, :],
                                  jnp.full((16,), k & 15, jnp.int32))
                for c in range(8):
                    d = offb - float(c)
                    ind = jnp.maximum(0.0, 1.0 - d * d)
                    stg_v[k, pl.ds(c * 16, 16)] = w * ind
                return 0

            lax.fori_loop(0, _CHB, row, 0, unroll=2)
            pltpu.sync_copy(w_v, w_h.at[pl.ds(off, _CHB)])
            pltpu.sync_copy(stg_v, s1_sh.at[dip_v], add=True)
            return 0

        lax.fori_loop(0, _EPT // _CHB, chunk, 0)
        plsc.subcore_barrier()
        pltpu.sync_copy(s1_sh.at[pl.ds(sid * nps, nps)],
                        s1_h.at[cid, pl.ds(sid * nps, nps)])

    return kb(src, dst, U, m1t, zb)


_CHC = 128


def _sc_c(src, dst, w1e, h2d, zc):
    mesh = plsc.VectorSubcoreMesh(core_axis_name="c", subcore_axis_name="s")

    @functools.partial(
        pl.kernel, mesh=mesh,
        out_type=jax.ShapeDtypeStruct((NBLK, NP, 128), jnp.float32),
        scratch_types=[
            pltpu.VMEM((_CHC,), jnp.int32),
            pltpu.VMEM((_CHC,), jnp.int32),
            pltpu.VMEM((_CHC, 128), jnp.float32),
            pltpu.VMEM((16, 128), jnp.float32),
            pltpu.VMEM_SHARED((NP, 128), jnp.float32),
            pltpu.SemaphoreType.DMA,
        ],
    )
    def kc(src_h, dst_h, w_h, h_h, zc_h, agg_h,
           si_v, di_v, rb_v, w_v, acc_sh, sem):
        cid = lax.axis_index("c")
        sid = lax.axis_index("s")
        ept = EPAD // 16
        base = sid * ept

        def block(bi, _):
            blk = cid * 8 + bi
            head = blk // 2
            boff = blk * NP
            pltpu.sync_copy(zc_h, acc_sh.at[pl.ds(sid * NTS, NTS)])
            plsc.subcore_barrier()

            def chunk(ci, _):
                off = base + ci * _CHC
                pltpu.sync_copy(src_h.at[pl.ds(off, _CHC)], si_v)
                pltpu.sync_copy(dst_h.at[pl.ds(off, _CHC)], di_v)
                rs = pl.multiple_of(lax.shift_right_logical(off, 3), 16)
                pltpu.sync_copy(w_h.at[pl.ds(rs, 16)], w_v)

                def fix(g, _):
                    si_v[pl.ds(g * 16, 16)] = si_v[pl.ds(g * 16, 16)] + boff
                    return 0

                lax.fori_loop(0, _CHC // 16, fix, 0, unroll=4)
                pltpu.async_copy(h_h.at[si_v], rb_v, sem).wait()

                def row(k, _):
                    wrow = w_v[k >> 3, pl.ds((k & 7) * 16, 16)]
                    w = _dyngather(wrow, jnp.full((16,), head, jnp.int32))
                    for c in range(8):
                        rb_v[k, pl.ds(c * 16, 16)] = (
                            rb_v[k, pl.ds(c * 16, 16)] * w)
                    return 0

                lax.fori_loop(0, _CHC, row, 0, unroll=2)
                pltpu.sync_copy(rb_v, acc_sh.at[di_v], add=True)
                return 0

            lax.fori_loop(0, ept // _CHC, chunk, 0)
            plsc.subcore_barrier()
            pltpu.sync_copy(acc_sh.at[pl.ds(sid * NTS, NTS)],
                            agg_h.at[blk, pl.ds(sid * NTS, NTS)])
            plsc.subcore_barrier()
            return 0

        lax.fori_loop(0, NBLK // 2, block, 0)

    return kc(src, dst, w1e, h2d, zc)


def _kf_body(h2c_ref, s2p_ref, m2_ref, b2_ref, outn_ref, s2t_ref):
    t = h2c_ref[...]
    l = t[:, 2:3] + t[:, 3:4]
    l = jnp.where(l > 0, l, 0.2 * l)
    wself = jnp.exp(l - m2_ref[0:1, 0:1])
    agg0 = s2p_ref[0, :, 0:1] + s2p_ref[1, :, 0:1]
    agg1 = s2p_ref[0, :, 1:2] + s2p_ref[1, :, 1:2]
    s2sum = s2p_ref[0, :, 2:3] + s2p_ref[1, :, 2:3]
    s2f = s2sum + wself
    r = 1.0 / s2f
    o0 = (agg0 + wself * t[:, 0:1]) * r + b2_ref[0:1, 0:1]
    o1 = (agg1 + wself * t[:, 1:2]) * r + b2_ref[0:1, 1:2]
    atts = wself * r
    il = lax.broadcasted_iota(jnp.int32, (1, 16), 1)
    outn_ref[...] = jnp.where(
        il == 0, o0, jnp.where(il == 1, o1, jnp.where(il == 2, atts, 0.0)))
    s2t_ref[...] = jnp.broadcast_to(s2f, (BN, 128))


def _tc_f(h2cat, s2p, m2bc, b2t):
    return pl.pallas_call(
        _kf_body,
        grid=(NP // BN,),
        in_specs=[
            pl.BlockSpec((BN, 16), lambda i: (i, 0)),
            pl.BlockSpec((2, BN, 16), lambda i: (0, i, 0)),
            pl.BlockSpec((1, 16), lambda i: (0, 0)),
            pl.BlockSpec((1, 16), lambda i: (0, 0)),
        ],
        out_specs=[
            pl.BlockSpec((BN, 16), lambda i: (i, 0)),
            pl.BlockSpec((BN, 128), lambda i: (i, 0)),
        ],
        out_shape=[
            jax.ShapeDtypeStruct((NP, 16), jnp.float32),
            jax.ShapeDtypeStruct((NP, 128), jnp.float32),
        ],
    )(h2cat, s2p, m2bc, b2t)


def _sc_e1(src, dst, Ts, Td, m2bc, cst, zb):
    mesh = plsc.VectorSubcoreMesh(core_axis_name="c", subcore_axis_name="s")

    @functools.partial(
        pl.kernel, mesh=mesh,
        out_type=[jax.ShapeDtypeStruct((EPAD, 16), jnp.float32),
                  jax.ShapeDtypeStruct((2, NP // 8, 128), jnp.float32)],
        scratch_types=[
            pltpu.VMEM((_CHB,), jnp.int32),
            pltpu.VMEM((_CHB,), jnp.int32),
            pltpu.VMEM((_CHB,), jnp.int32),
            pltpu.VMEM((_CHB // 16, 16), jnp.float32),
            pltpu.VMEM((_CHB, 128), jnp.float32),
            pltpu.VMEM((_CHB, 128), jnp.float32),
            pltpu.VMEM((_CHB, 16), jnp.float32),
            pltpu.VMEM((_CHB, 128), jnp.float32),
            pltpu.VMEM((16,), jnp.float32),
            pltpu.VMEM((32,), jnp.float32),
            pltpu.VMEM_SHARED((NP // 8, 128), jnp.float32),
            pltpu.SemaphoreType.DMA,
        ],
    )
    def ke1(src_h, dst_h, ts_h, td_h, m2_h, cst_h, zb_h, attw_h, s2_h,
            si_v, di_v, dip_v, off_v, gs_v, gd_v, w_v, stg_v, m2_v, c_v,
            acc_sh, sem):
        cid = lax.axis_index("c")
        sid = lax.axis_index("s")
        wid = sid * 2 + cid
        nps = NP // 128
        pltpu.sync_copy(zb_h, acc_sh.at[pl.ds(sid * nps, nps)])
        pltpu.sync_copy(m2_h.at[0], m2_v)
        pltpu.sync_copy(cst_h, c_v)
        plsc.subcore_barrier()
        m2 = m2_v[...]
        mlo = c_v[pl.ds(0, 16)]
        ms2 = c_v[pl.ds(16, 16)]
        c2 = jnp.full((16,), 2, jnp.int32)
        base = wid * _EPT

        def chunk(ci, _):
            off = base + ci * _CHB
            pltpu.sync_copy(src_h.at[pl.ds(off, _CHB)], si_v)
            pltpu.sync_copy(dst_h.at[pl.ds(off, _CHB)], di_v)
            pltpu.async_copy(ts_h.at[si_v], gs_v, sem).wait()
            pltpu.async_copy(td_h.at[di_v], gd_v, sem).wait()

            def pre(g, _):
                dvec = di_v[pl.ds(g * 16, 16)]
                off_v[g, :] = (dvec & 7).astype(jnp.float32)
                dip_v[pl.ds(g * 16, 16)] = lax.shift_right_logical(dvec, 3)
                return 0

            lax.fori_loop(0, _CHB // 16, pre, 0)

            def row(k, _):
                srow = gs_v[k, pl.ds(0, 16)]
                drow = gd_v[k, pl.ds(0, 16)]
                l = srow + drow
                l = jnp.where(l > 0, l, 0.2 * l)
                w_all = jnp.exp(l - m2)
                w2 = _dyngather(w_all, c2)
                contrib = w2 * (srow * mlo + ms2)
                w_v[k, :] = contrib
                offb = _dyngather(off_v[k >> 4, :],
                                  jnp.full((16,), k & 15, jnp.int32))
                for c in range(8):
                    d = offb - float(c)
                    ind = jnp.maximum(0.0, 1.0 - d * d)
                    stg_v[k, pl.ds(c * 16, 16)] = contrib * ind
                return 0

            lax.fori_loop(0, _CHB, row, 0, unroll=2)
            pltpu.sync_copy(w_v, attw_h.at[pl.ds(off, _CHB)])
            pltpu.sync_copy(stg_v, acc_sh.at[dip_v], add=True)
            return 0

        lax.fori_loop(0, _EPT // _CHB, chunk, 0)
        plsc.subcore_barrier()
        pltpu.sync_copy(acc_sh.at[pl.ds(sid * nps, nps)],
                        s2_h.at[cid, pl.ds(sid * nps, nps)])

    return ke1(src, dst, Ts, Td, m2bc, cst, zb)


def _sc_e2(dst, attw, S2T):
    mesh = plsc.VectorSubcoreMesh(core_axis_name="c", subcore_axis_name="s")

    @functools.partial(
        pl.kernel, mesh=mesh,
        out_type=jax.ShapeDtypeStruct((EPAD, 16), jnp.float32),
        scratch_types=[
            pltpu.VMEM((_CHB,), jnp.int32),
            pltpu.VMEM((_CHB, 16), jnp.float32),
            pltpu.VMEM((_CHB, 128), jnp.float32),
            pltpu.SemaphoreType.DMA,
        ],
    )
    def ke2(dst_h, w_h, s2t_h, attf_h, di_v, wv_v, gv_v, sem):
        cid = lax.axis_index("c")
        sid = lax.axis_index("s")
        wid = sid * 2 + cid
        base = wid * _EPT

        def chunk(ci, _):
            off = base + ci * _CHB
            pltpu.sync_copy(dst_h.at[pl.ds(off, _CHB)], di_v)
            pltpu.sync_copy(w_h.at[pl.ds(off, _CHB)], wv_v)
            pltpu.async_copy(s2t_h.at[di_v], gv_v, sem).wait()

            def row(k, _):
                wv_v[k, :] = wv_v[k, :] / gv_v[k, pl.ds(0, 16)]
                return 0

            lax.fori_loop(0, _CHB, row, 0, unroll=4)
            pltpu.sync_copy(wv_v, attf_h.at[pl.ds(off, _CHB)])
            return 0

        lax.fori_loop(0, _EPT // _CHB, chunk, 0)

    return ke2(dst, attw, S2T)


def kernel(x, edge_index, tconv_w, tconv_b, bn0_w, bn0_b, bn0_rm, bn0_rv,
           W1, att_src1, att_dst1, bias1, bn1_w, bn1_b, bn1_rm, bn1_rv,
           W2, att_src2, att_dst2, bias2):
    f32 = jnp.float32
    wi = jnp.arange(WIN)[:, None]
    wo = jnp.arange(WIN)[None, :]
    kk = jnp.arange(3)[:, None, None]
    band = (wi[None] == wo[None] + kk - 1).astype(f32)
    Wc = jnp.einsum('tfk,kio->fito', tconv_w, band).reshape(IN_DIM, TC_CH * WIN)
    b0r = jnp.repeat(tconv_b, WIN)[None, :]
    g0 = bn0_w / jnp.sqrt(bn0_rv + EPS)
    c0 = bn0_b - bn0_rm * g0
    W1g = W1 * g0[:, None]
    d1 = (c0 @ W1)[None, :]
    rows = jnp.arange(HEADS * HID)
    i16 = jnp.arange(16)
    headmask = ((rows // HID)[:, None] == (i16[None, :] % 8)).astype(f32)
    As2 = att_src1.reshape(-1)[:, None] * headmask
    Ad2 = att_dst1.reshape(-1)[:, None] * headmask
    AU = jnp.concatenate(
        [As2, Ad2, jnp.zeros((HEADS * HID, 96), f32)], axis=1)
    g1 = bn1_w / jnp.sqrt(bn1_rv + EPS)
    c1 = (bias1 - bn1_rm) * g1 + bn1_b
    g1b = g1.reshape(NBLK, 1, 128)
    c1b = c1.reshape(NBLK, 1, 128)
    v_s = W2 @ att_src2[0]
    v_d = W2 @ att_dst2[0]
    W2ext = jnp.zeros((HEADS * HID, 16), f32)
    W2ext = W2ext.at[:, 0:2].set(W2)
    W2ext = W2ext.at[:, 2].set(v_s)
    W2ext = W2ext.at[:, 3].set(v_d)
    M16b = jnp.broadcast_to(
        ((i16 == 2) | (i16 == 3))[:, None], (16, 16)).astype(f32)
    b2t = jnp.zeros((1, 16), f32)
    b2t = b2t.at[0, 0].set(bias2[0]).at[0, 1].set(bias2[1])
    zb = jnp.zeros((NP // 128, 128), f32)
    zc = jnp.zeros((NTS, 128), f32)

    xp = jnp.pad(x, ((0, NP - N), (0, 0)))
    padv = (jnp.arange(EPAD - E, dtype=jnp.int32) % (NP - N)) + N
    src = jnp.concatenate([edge_index[0], padv])
    dst = jnp.concatenate([edge_index[1], padv])

    hb, U, m1t = _tc_a(xp, Wc, b0r, W1g, d1, AU)

    w1e, s1p = _sc_b(src, dst, U, m1t, zb)

    h2d = hb.reshape(NBLK * NP, 128)
    w2d = w1e.reshape(EPAD // 8, 128)
    aggb = _sc_c(src, dst, w2d, h2d, zc)

    s1pu = s1p.reshape(2, NP, 16)
    h2cat, m2bc, Ts, Td = _tc_d(hb, aggb, U, s1pu, m1t, g1b, c1b,
                                W2ext, M16b)

    cst = jnp.concatenate([
        (jnp.arange(16) < 2).astype(f32),
        (jnp.arange(16) == 2).astype(f32)])

    attw, s2p = _sc_e1(src, dst, Ts, Td, m2bc, cst, zb)

    outn, S2T = _tc_f(h2cat, s2p.reshape(2, NP, 16), m2bc, b2t)

    attf = _sc_e2(dst, attw, S2T)

    out = outn[:N, 0:2]
    att2 = jnp.concatenate([attf[:E, 2], outn[:N, 2]])[:, None]
    return (out, att2)
